# Initial kernel scaffold; baseline (speedup 1.0000x reference)
import jax
import jax.numpy as jnp
from jax.experimental import pallas as pl


def kernel(x, gate_w, ew1, ew2, ew3, sw1, sw2, sw3):
    raise NotImplementedError("write your pallas kernel here")


# speedup vs baseline: 1.3835x; 1.3835x over previous
import functools

import jax
import jax.numpy as jnp
from jax.experimental import pallas as pl
from jax.experimental.pallas import tpu as pltpu

AUX_COEF = 0.01
Z_COEF = 0.001


def _router_kernel(x_ref, gw_ref, wt_ref, loss_ref):
    logits = jax.lax.dot_general(
        gw_ref[...], x_ref[...], (((1,), (1,)), ((), ())),
        preferred_element_type=jnp.float32)
    E = logits.shape[0]
    T = logits.shape[1]
    m = jnp.max(logits, axis=0, keepdims=True)
    ex = jnp.exp(logits - m)
    ssum = jnp.sum(ex, axis=0, keepdims=True)
    s = ex / ssum
    lse = m + jnp.log(ssum)
    z_loss = Z_COEF * jnp.sum(lse * lse) / T

    iota = jax.lax.broadcasted_iota(jnp.int32, (E, T), 0)
    big = jnp.int32(E)
    m1 = jnp.max(s, axis=0, keepdims=True)
    idx0 = jnp.min(jnp.where(s == m1, iota, big), axis=0, keepdims=True)
    sel0 = iota == idx0
    s2 = jnp.where(sel0, -1.0, s)
    m2 = jnp.max(s2, axis=0, keepdims=True)
    idx1 = jnp.min(jnp.where(s2 == m2, iota, big), axis=0, keepdims=True)
    sel1 = iota == idx1
    selb = jnp.logical_or(sel0, sel1)
    wt_ref[...] = jnp.where(selb, s, 0.0)

    counts = jnp.sum(selb.astype(jnp.float32), axis=1, keepdims=True)
    p_mean = jnp.sum(s, axis=1, keepdims=True) / T
    aux = AUX_COEF * E * jnp.sum(counts / T * p_mean)
    loss_ref[...] = jnp.reshape(aux + z_loss, (1, 1))


def _routed_kernel(x_ref, w_ref, w1_ref, w2_ref, w3_ref, o_ref, xb_ref):
    e = pl.program_id(1)
    c = pl.program_id(2)

    @pl.when(jnp.logical_and(e == 0, c == 0))
    def _():
        o_ref[...] = jnp.zeros_like(o_ref)
        xb_ref[...] = x_ref[...].astype(jnp.bfloat16)

    xb = xb_ref[...]
    w1 = w1_ref[0].astype(jnp.bfloat16)
    w2 = w2_ref[0].astype(jnp.bfloat16)
    w3 = w3_ref[0].astype(jnp.bfloat16)
    a = jax.lax.dot_general(xb, w1, (((1,), (1,)), ((), ())),
                            preferred_element_type=jnp.float32)
    b = jax.lax.dot_general(xb, w2, (((1,), (1,)), ((), ())),
                            preferred_element_type=jnp.float32)
    h = a * jax.nn.sigmoid(a) * b
    lane = jax.lax.broadcasted_iota(jnp.int32, w_ref.shape, 1)
    wcol = jnp.sum(jnp.where(lane == e, w_ref[...], 0.0), axis=1,
                   keepdims=True)
    hw = (h * wcol).astype(jnp.bfloat16)
    o_ref[...] += jax.lax.dot_general(hw, w3, (((1,), (1,)), ((), ())),
                                      preferred_element_type=jnp.float32)


def _shared_kernel(x_ref, w1_ref, w2_ref, w3_ref, o_ref, xb_ref):
    j = pl.program_id(1)
    c = pl.program_id(2)

    @pl.when(jnp.logical_and(j == 0, c == 0))
    def _():
        o_ref[...] = jnp.zeros_like(o_ref)
        xb_ref[...] = x_ref[...].astype(jnp.bfloat16)

    xb = xb_ref[...]
    w1 = w1_ref[0].astype(jnp.bfloat16)
    w2 = w2_ref[0].astype(jnp.bfloat16)
    w3 = w3_ref[0].astype(jnp.bfloat16)
    a = jax.lax.dot_general(xb, w1, (((1,), (1,)), ((), ())),
                            preferred_element_type=jnp.float32)
    b = jax.lax.dot_general(xb, w2, (((1,), (1,)), ((), ())),
                            preferred_element_type=jnp.float32)
    h = (a * jax.nn.sigmoid(a) * b).astype(jnp.bfloat16)
    o_ref[...] += jax.lax.dot_general(h, w3, (((1,), (1,)), ((), ())),
                                      preferred_element_type=jnp.float32)


@functools.partial(jax.jit, static_argnames=())
def kernel(x, gate_w, ew1, ew2, ew3, sw1, sw2, sw3):
    B, S, D = x.shape
    T = B * S
    E, DFF, _ = ew1.shape
    NS, DFH, _ = sw1.shape
    x_flat = x.reshape(T, D)

    wt, loss = pl.pallas_call(
        _router_kernel,
        out_shape=[
            jax.ShapeDtypeStruct((E, T), jnp.float32),
            jax.ShapeDtypeStruct((1, 1), jnp.float32),
        ],
    )(x_flat, gate_w)
    wfull = jnp.pad(wt.T, ((0, 0), (0, 128 - E)))

    TH = T // 2
    FC = min(1024, DFF)
    NC = DFF // FC
    routed = pl.pallas_call(
        _routed_kernel,
        grid=(2, E, NC),
        in_specs=[
            pl.BlockSpec((TH, D), lambda h, e, c: (h, 0)),
            pl.BlockSpec((TH, 128), lambda h, e, c: (h, 0)),
            pl.BlockSpec((1, FC, D), lambda h, e, c: (e, c, 0)),
            pl.BlockSpec((1, FC, D), lambda h, e, c: (e, c, 0)),
            pl.BlockSpec((1, D, FC), lambda h, e, c: (e, 0, c)),
        ],
        out_specs=pl.BlockSpec((TH, D), lambda h, e, c: (h, 0)),
        out_shape=jax.ShapeDtypeStruct((T, D), jnp.float32),
        scratch_shapes=[pltpu.VMEM((TH, D), jnp.bfloat16)],
        compiler_params=pltpu.CompilerParams(
            dimension_semantics=("parallel", "arbitrary", "arbitrary")),
    )(x_flat, wfull, ew1, ew2, ew3)

    FCS = min(1024, DFH)
    NCS = DFH // FCS
    shared = pl.pallas_call(
        _shared_kernel,
        grid=(2, NS, NCS),
        in_specs=[
            pl.BlockSpec((TH, D), lambda h, j, c: (h, 0)),
            pl.BlockSpec((1, FCS, D), lambda h, j, c: (j, c, 0)),
            pl.BlockSpec((1, FCS, D), lambda h, j, c: (j, c, 0)),
            pl.BlockSpec((1, D, FCS), lambda h, j, c: (j, 0, c)),
        ],
        out_specs=pl.BlockSpec((TH, D), lambda h, j, c: (h, 0)),
        out_shape=jax.ShapeDtypeStruct((T, D), jnp.float32),
        scratch_shapes=[pltpu.VMEM((TH, D), jnp.bfloat16)],
        compiler_params=pltpu.CompilerParams(
            dimension_semantics=("parallel", "arbitrary", "arbitrary")),
    )(x_flat, sw1, sw2, sw3)

    out = routed + shared
    return out.reshape(B, S, D), loss[0, 0]


# speedup vs baseline: 1.6175x; 1.1691x over previous
import functools

import jax
import jax.numpy as jnp
from jax import lax
from jax.experimental import pallas as pl
from jax.experimental.pallas import tpu as pltpu
from jax.experimental.pallas import tpu_sc as plsc

AUX_COEF = 0.01
Z_COEF = 0.001
BT = 256


def _router_kernel(x_ref, gw_ref, ws_ref, dest_ref, bexp_ref, nlive_ref,
                   loss_ref):
    nbmax = bexp_ref.shape[1]
    logits = lax.dot_general(gw_ref[...], x_ref[...], (((1,), (1,)), ((), ())),
                             preferred_element_type=jnp.float32)
    E, T = logits.shape
    m = jnp.max(logits, axis=0, keepdims=True)
    ex = jnp.exp(logits - m)
    ssum = jnp.sum(ex, axis=0, keepdims=True)
    s = ex / ssum
    lse = m + jnp.log(ssum)
    z_loss = Z_COEF * jnp.sum(lse * lse) / T

    iota = lax.broadcasted_iota(jnp.int32, (E, T), 0)
    big = jnp.int32(E)
    m1 = jnp.max(s, axis=0, keepdims=True)
    idx0 = jnp.min(jnp.where(s == m1, iota, big), axis=0, keepdims=True)
    sel0 = iota == idx0
    s2 = jnp.where(sel0, -1.0, s)
    m2 = jnp.max(s2, axis=0, keepdims=True)
    idx1 = jnp.min(jnp.where(s2 == m2, iota, big), axis=0, keepdims=True)
    sel1 = iota == idx1
    selb = jnp.logical_or(sel0, sel1)
    self_ = selb.astype(jnp.float32)

    ws_ref[0:1, :] = jnp.sum(jnp.where(sel0, s, 0.0), axis=0, keepdims=True)
    ws_ref[1:2, :] = jnp.sum(jnp.where(sel1, s, 0.0), axis=0, keepdims=True)

    counts = jnp.sum(self_, axis=1, keepdims=True)
    p_mean = jnp.sum(s, axis=1, keepdims=True) / T
    aux = AUX_COEF * E * jnp.sum(counts / T * p_mean)
    loss_ref[...] = jnp.reshape(aux + z_loss, (1, 1))

    cum = jnp.concatenate(
        [jnp.zeros((E, 1), jnp.float32), self_[:, :-1]], axis=1)
    k = 1
    while k < T:
        cum = cum + jnp.concatenate(
            [jnp.zeros((E, k), jnp.float32), cum[:, :-k]], axis=1)
        k *= 2

    pad_c = jnp.floor((counts + (BT - 1)) * (1.0 / BT)) * BT
    tri = (lax.broadcasted_iota(jnp.int32, (E, E), 1)
           < lax.broadcasted_iota(jnp.int32, (E, E), 0)).astype(jnp.float32)
    po = lax.dot_general(tri, pad_c, (((1,), (0,)), ((), ())),
                         preferred_element_type=jnp.float32)

    rowf = po + cum
    d0 = jnp.sum(jnp.where(sel0, rowf, 0.0), axis=0, keepdims=True)
    d1 = jnp.sum(jnp.where(sel1, rowf, 0.0), axis=0, keepdims=True)
    dest_ref[0:1, :] = d0.astype(jnp.int32)
    dest_ref[1:2, :] = d1.astype(jnp.int32)

    brow = lax.broadcasted_iota(jnp.int32, (E, 128), 1) * BT
    po_i = po.astype(jnp.int32)
    bexp = jnp.sum((po_i <= brow).astype(jnp.int32), axis=0, keepdims=True) - 1
    bexp_ref[...] = bexp[:, :nbmax]
    nlive_ref[...] = (jnp.sum(pad_c, keepdims=True) *
                      (1.0 / BT)).astype(jnp.int32)


def _grouped_kernel(sarr_ref, x_ref, dest_ref, w1_ref, w2_ref, w3_ref,
                    yg_ref, xg_ref):
    c = pl.program_id(0)
    b = pl.program_id(1)
    nlive = sarr_ref[sarr_ref.shape[0] - 1]
    base = b * BT

    @pl.when(b < nlive)
    def _():
        @pl.when(c == 0)
        def _():
            rid = base + lax.broadcasted_iota(jnp.int32,
                                              (BT, dest_ref.shape[1]), 0)
            eq = jnp.logical_or(rid == dest_ref[0:1, :],
                                rid == dest_ref[1:2, :])
            mb = eq.astype(jnp.bfloat16)
            xg_ref[pl.ds(base, BT), :] = lax.dot_general(
                mb, x_ref[...], (((1,), (0,)), ((), ())),
                preferred_element_type=jnp.float32).astype(jnp.bfloat16)

        xgb = xg_ref[pl.ds(base, BT), :]
        w1 = w1_ref[0].astype(jnp.bfloat16)
        w2 = w2_ref[0].astype(jnp.bfloat16)
        w3 = w3_ref[0].astype(jnp.bfloat16)
        a = lax.dot_general(xgb, w1, (((1,), (1,)), ((), ())),
                            preferred_element_type=jnp.float32)
        bb = lax.dot_general(xgb, w2, (((1,), (1,)), ((), ())),
                             preferred_element_type=jnp.float32)
        h = (a * jax.nn.sigmoid(a) * bb).astype(jnp.bfloat16)
        contrib = lax.dot_general(h, w3, (((1,), (1,)), ((), ())),
                                  preferred_element_type=jnp.float32)

        @pl.when(c == 0)
        def _():
            yg_ref[pl.ds(base, BT), :] = contrib

        @pl.when(c > 0)
        def _():
            yg_ref[pl.ds(base, BT), :] += contrib


def _shared_combine_kernel(x_ref, w1_ref, w2_ref, w3_ref, g0_ref, g1_ref,
                           wc_ref, o_ref):
    j = pl.program_id(1)
    c = pl.program_id(2)
    nj = pl.num_programs(1)
    nc = pl.num_programs(2)

    xb = x_ref[...]
    w1 = w1_ref[0].astype(jnp.bfloat16)
    w2 = w2_ref[0].astype(jnp.bfloat16)
    w3 = w3_ref[0].astype(jnp.bfloat16)
    a = lax.dot_general(xb, w1, (((1,), (1,)), ((), ())),
                        preferred_element_type=jnp.float32)
    bb = lax.dot_general(xb, w2, (((1,), (1,)), ((), ())),
                         preferred_element_type=jnp.float32)
    h = (a * jax.nn.sigmoid(a) * bb).astype(jnp.bfloat16)
    contrib = lax.dot_general(h, w3, (((1,), (1,)), ((), ())),
                              preferred_element_type=jnp.float32)

    @pl.when(jnp.logical_and(j == 0, c == 0))
    def _():
        o_ref[...] = contrib

    @pl.when(jnp.logical_not(jnp.logical_and(j == 0, c == 0)))
    def _():
        o_ref[...] += contrib

    @pl.when(jnp.logical_and(j == nj - 1, c == nc - 1))
    def _():
        lane = lax.broadcasted_iota(jnp.int32, wc_ref.shape, 1)
        w0c = jnp.sum(jnp.where(lane == 0, wc_ref[...], 0.0), axis=1,
                      keepdims=True)
        w1c = jnp.sum(jnp.where(lane == 1, wc_ref[...], 0.0), axis=1,
                      keepdims=True)
        o_ref[...] += w0c * g0_ref[...] + w1c * g1_ref[...]


def _dispatch_gather(yg, dest_flat):
    nr, d = yg.shape
    n = dest_flat.shape[0]
    info = plsc.get_sparse_core_info()
    ncores, nsub = info.num_cores, info.num_subcores
    nw = ncores * nsub
    per_w = n // nw
    chunk = 32
    mesh = plsc.VectorSubcoreMesh(core_axis_name="c", subcore_axis_name="s")

    @functools.partial(
        pl.kernel, mesh=mesh,
        out_type=jax.ShapeDtypeStruct((n, d), jnp.float32),
        scratch_types=[
            pltpu.VMEM((chunk,), jnp.int32),
            pltpu.VMEM((chunk, d), jnp.float32),
            pltpu.SemaphoreType.DMA,
        ],
    )
    def sc_gather(yg_hbm, idx_hbm, g_hbm, idx_v, rows_v, sem):
        wid = lax.axis_index("s") * ncores + lax.axis_index("c")
        base = wid * per_w
        for ch in range(per_w // chunk):
            off = base + ch * chunk
            pltpu.sync_copy(idx_hbm.at[pl.ds(off, chunk)], idx_v)
            pltpu.async_copy(yg_hbm.at[idx_v], rows_v, sem).wait()
            pltpu.sync_copy(rows_v, g_hbm.at[pl.ds(off, chunk)])

    return sc_gather(yg, dest_flat)


@jax.jit
def kernel(x, gate_w, ew1, ew2, ew3, sw1, sw2, sw3):
    B, S, D = x.shape
    T = B * S
    E, DFF, _ = ew1.shape
    NS, DFH, _ = sw1.shape
    x_flat = x.reshape(T, D)

    NB = (2 * T + E * (BT - 1)) // BT
    NR = NB * BT

    ws, dest, bexp, nlive, loss = pl.pallas_call(
        _router_kernel,
        out_shape=[
            jax.ShapeDtypeStruct((2, T), jnp.float32),
            jax.ShapeDtypeStruct((2, T), jnp.int32),
            jax.ShapeDtypeStruct((1, NB), jnp.int32),
            jax.ShapeDtypeStruct((1, 1), jnp.int32),
            jax.ShapeDtypeStruct((1, 1), jnp.float32),
        ],
    )(x_flat, gate_w)

    sarr = jnp.concatenate([bexp.reshape(NB), nlive.reshape(1)])
    xb16 = x_flat.astype(jnp.bfloat16)

    FC = min(512, DFF)
    NC = DFF // FC
    yg = pl.pallas_call(
        _grouped_kernel,
        grid_spec=pltpu.PrefetchScalarGridSpec(
            num_scalar_prefetch=1,
            grid=(NC, NB),
            in_specs=[
                pl.BlockSpec((T, D), lambda c, b, sarr: (0, 0)),
                pl.BlockSpec((2, T), lambda c, b, sarr: (0, 0)),
                pl.BlockSpec((1, FC, D), lambda c, b, sarr: (sarr[b], c, 0)),
                pl.BlockSpec((1, FC, D), lambda c, b, sarr: (sarr[b], c, 0)),
                pl.BlockSpec((1, D, FC), lambda c, b, sarr: (sarr[b], 0, c)),
            ],
            out_specs=pl.BlockSpec((NR, D), lambda c, b, sarr: (0, 0)),
            scratch_shapes=[pltpu.VMEM((NR, D), jnp.bfloat16)],
        ),
        out_shape=jax.ShapeDtypeStruct((NR, D), jnp.float32),
        compiler_params=pltpu.CompilerParams(
            dimension_semantics=("arbitrary", "arbitrary")),
    )(sarr, xb16, dest, ew1, ew2, ew3)

    g = _dispatch_gather(yg, dest.reshape(2 * T))

    wcols = jnp.pad(ws.T, ((0, 0), (0, 126)))

    TH = T // 2
    FCS = min(512, DFH)
    NCS = DFH // FCS
    out = pl.pallas_call(
        _shared_combine_kernel,
        grid=(2, NS, NCS),
        in_specs=[
            pl.BlockSpec((TH, D), lambda h, j, c: (h, 0)),
            pl.BlockSpec((1, FCS, D), lambda h, j, c: (j, c, 0)),
            pl.BlockSpec((1, FCS, D), lambda h, j, c: (j, c, 0)),
            pl.BlockSpec((1, D, FCS), lambda h, j, c: (j, 0, c)),
            pl.BlockSpec((TH, D), lambda h, j, c: (h, 0)),
            pl.BlockSpec((TH, D), lambda h, j, c: (h + 2, 0)),
            pl.BlockSpec((TH, 128), lambda h, j, c: (h, 0)),
        ],
        out_specs=pl.BlockSpec((TH, D), lambda h, j, c: (h, 0)),
        out_shape=jax.ShapeDtypeStruct((T, D), jnp.float32),
        compiler_params=pltpu.CompilerParams(
            dimension_semantics=("parallel", "arbitrary", "arbitrary")),
    )(xb16, sw1, sw2, sw3, g, g, wcols)

    return out.reshape(B, S, D), loss[0, 0]


# speedup vs baseline: 1.7145x; 1.0600x over previous
import functools

import jax
import jax.numpy as jnp
from jax import lax
from jax.experimental import pallas as pl
from jax.experimental.pallas import tpu as pltpu
from jax.experimental.pallas import tpu_sc as plsc

AUX_COEF = 0.01
Z_COEF = 0.001
BT = 256


def _router_kernel(x_ref, gw_ref, ws_ref, dest_ref, bexp_ref, nlive_ref,
                   loss_ref, xg_ref):
    nbmax = bexp_ref.shape[1]
    logits = lax.dot_general(gw_ref[...], x_ref[...], (((1,), (1,)), ((), ())),
                             preferred_element_type=jnp.float32)
    E, T = logits.shape
    m = jnp.max(logits, axis=0, keepdims=True)
    ex = jnp.exp(logits - m)
    ssum = jnp.sum(ex, axis=0, keepdims=True)
    s = ex / ssum
    lse = m + jnp.log(ssum)
    z_loss = Z_COEF * jnp.sum(lse * lse) / T

    iota = lax.broadcasted_iota(jnp.int32, (E, T), 0)
    big = jnp.int32(E)
    m1 = jnp.max(s, axis=0, keepdims=True)
    idx0 = jnp.min(jnp.where(s == m1, iota, big), axis=0, keepdims=True)
    sel0 = iota == idx0
    s2 = jnp.where(sel0, -1.0, s)
    m2 = jnp.max(s2, axis=0, keepdims=True)
    idx1 = jnp.min(jnp.where(s2 == m2, iota, big), axis=0, keepdims=True)
    sel1 = iota == idx1
    selb = jnp.logical_or(sel0, sel1)
    self_ = selb.astype(jnp.float32)

    ws_ref[0:1, :] = jnp.sum(jnp.where(sel0, s, 0.0), axis=0, keepdims=True)
    ws_ref[1:2, :] = jnp.sum(jnp.where(sel1, s, 0.0), axis=0, keepdims=True)

    counts = jnp.sum(self_, axis=1, keepdims=True)
    p_mean = jnp.sum(s, axis=1, keepdims=True) / T
    aux = AUX_COEF * E * jnp.sum(counts / T * p_mean)
    loss_ref[...] = jnp.reshape(aux + z_loss, (1, 1))

    cum = jnp.concatenate(
        [jnp.zeros((E, 1), jnp.float32), self_[:, :-1]], axis=1)
    k = 1
    while k < T:
        cum = cum + jnp.concatenate(
            [jnp.zeros((E, k), jnp.float32), cum[:, :-k]], axis=1)
        k *= 2

    pad_c = jnp.floor((counts + (BT - 1)) * (1.0 / BT)) * BT
    tri = (lax.broadcasted_iota(jnp.int32, (E, E), 1)
           < lax.broadcasted_iota(jnp.int32, (E, E), 0)).astype(jnp.float32)
    po = lax.dot_general(tri, pad_c, (((1,), (0,)), ((), ())),
                         preferred_element_type=jnp.float32)

    rowf = po + cum
    d0 = jnp.sum(jnp.where(sel0, rowf, 0.0), axis=0, keepdims=True)
    d1 = jnp.sum(jnp.where(sel1, rowf, 0.0), axis=0, keepdims=True)
    d0i = d0.astype(jnp.int32)
    d1i = d1.astype(jnp.int32)
    dest_ref[0:1, :] = d0i
    dest_ref[1:2, :] = d1i

    brow = lax.broadcasted_iota(jnp.int32, (E, 128), 1) * BT
    po_i = po.astype(jnp.int32)
    bexp = jnp.sum((po_i <= brow).astype(jnp.int32), axis=0, keepdims=True) - 1
    bexp_ref[...] = bexp[:, :nbmax]
    nlive_ref[...] = (jnp.sum(pad_c, keepdims=True) *
                      (1.0 / BT)).astype(jnp.int32)

    xb = x_ref[...].astype(jnp.bfloat16)
    for b in range(nbmax):
        rid = BT * b + lax.broadcasted_iota(jnp.int32, (BT, T), 0)
        eq = jnp.logical_or(rid == d0i, rid == d1i)
        mb = eq.astype(jnp.bfloat16)
        xg_ref[BT * b:BT * (b + 1), :] = lax.dot_general(
            mb, xb, (((1,), (0,)), ((), ())),
            preferred_element_type=jnp.float32).astype(jnp.bfloat16)


def _grouped_kernel(sarr_ref, xg_ref, w1_ref, w2_ref, w3_ref, yg_ref,
                    w1s_ref, w2s_ref, w3s_ref):
    c = pl.program_id(0)
    b = pl.program_id(1)
    nb = pl.num_programs(1)
    nlive = sarr_ref[nb]
    base = b * BT

    @pl.when(b < nlive)
    def _():
        prev = jnp.where(b == 0, -1, sarr_ref[jnp.maximum(b - 1, 0)])

        @pl.when(sarr_ref[b] != prev)
        def _():
            w1s_ref[...] = w1_ref[0].astype(jnp.bfloat16)
            w2s_ref[...] = w2_ref[0].astype(jnp.bfloat16)
            w3s_ref[...] = w3_ref[0].astype(jnp.bfloat16)

        xgb = xg_ref[...]
        a = lax.dot_general(xgb, w1s_ref[...], (((1,), (1,)), ((), ())),
                            preferred_element_type=jnp.float32)
        bb = lax.dot_general(xgb, w2s_ref[...], (((1,), (1,)), ((), ())),
                             preferred_element_type=jnp.float32)
        h = (a * jax.nn.sigmoid(a) * bb).astype(jnp.bfloat16)
        contrib = lax.dot_general(h, w3s_ref[...], (((1,), (1,)), ((), ())),
                                  preferred_element_type=jnp.float32)

        @pl.when(c == 0)
        def _():
            yg_ref[pl.ds(base, BT), :] = contrib

        @pl.when(c > 0)
        def _():
            yg_ref[pl.ds(base, BT), :] += contrib


def _shared_combine_kernel(x_ref, w1_ref, w2_ref, w3_ref, g0_ref, g1_ref,
                           wc_ref, o_ref):
    j = pl.program_id(1)
    c = pl.program_id(2)
    nj = pl.num_programs(1)
    nc = pl.num_programs(2)

    xb = x_ref[...]
    w1 = w1_ref[0].astype(jnp.bfloat16)
    w2 = w2_ref[0].astype(jnp.bfloat16)
    w3 = w3_ref[0].astype(jnp.bfloat16)
    a = lax.dot_general(xb, w1, (((1,), (1,)), ((), ())),
                        preferred_element_type=jnp.float32)
    bb = lax.dot_general(xb, w2, (((1,), (1,)), ((), ())),
                         preferred_element_type=jnp.float32)
    h = (a * jax.nn.sigmoid(a) * bb).astype(jnp.bfloat16)
    contrib = lax.dot_general(h, w3, (((1,), (1,)), ((), ())),
                              preferred_element_type=jnp.float32)

    @pl.when(jnp.logical_and(j == 0, c == 0))
    def _():
        o_ref[...] = contrib

    @pl.when(jnp.logical_not(jnp.logical_and(j == 0, c == 0)))
    def _():
        o_ref[...] += contrib

    @pl.when(jnp.logical_and(j == nj - 1, c == nc - 1))
    def _():
        lane = lax.broadcasted_iota(jnp.int32, wc_ref.shape, 1)
        w0c = jnp.sum(jnp.where(lane == 0, wc_ref[...], 0.0), axis=1,
                      keepdims=True)
        w1c = jnp.sum(jnp.where(lane == 1, wc_ref[...], 0.0), axis=1,
                      keepdims=True)
        o_ref[...] += w0c * g0_ref[...] + w1c * g1_ref[...]


def _dispatch_gather(yg, dest_flat):
    nr, d = yg.shape
    n = dest_flat.shape[0]
    info = plsc.get_sparse_core_info()
    ncores, nsub = info.num_cores, info.num_subcores
    nw = ncores * nsub
    per_w = n // nw
    chunk = 32
    mesh = plsc.VectorSubcoreMesh(core_axis_name="c", subcore_axis_name="s")

    @functools.partial(
        pl.kernel, mesh=mesh,
        out_type=jax.ShapeDtypeStruct((n, d), jnp.float32),
        scratch_types=[
            pltpu.VMEM((chunk,), jnp.int32),
            pltpu.VMEM((chunk, d), jnp.float32),
            pltpu.SemaphoreType.DMA,
        ],
    )
    def sc_gather(yg_hbm, idx_hbm, g_hbm, idx_v, rows_v, sem):
        wid = lax.axis_index("s") * ncores + lax.axis_index("c")
        base = wid * per_w
        for ch in range(per_w // chunk):
            off = base + ch * chunk
            pltpu.sync_copy(idx_hbm.at[pl.ds(off, chunk)], idx_v)
            pltpu.async_copy(yg_hbm.at[idx_v], rows_v, sem).wait()
            pltpu.sync_copy(rows_v, g_hbm.at[pl.ds(off, chunk)])

    return sc_gather(yg, dest_flat)


@jax.jit
def kernel(x, gate_w, ew1, ew2, ew3, sw1, sw2, sw3):
    B, S, D = x.shape
    T = B * S
    E, DFF, _ = ew1.shape
    NS, DFH, _ = sw1.shape
    x_flat = x.reshape(T, D)

    NB = (2 * T + E * (BT - 1)) // BT
    NR = NB * BT

    ws, dest, bexp, nlive, loss, xg = pl.pallas_call(
        _router_kernel,
        out_shape=[
            jax.ShapeDtypeStruct((2, T), jnp.float32),
            jax.ShapeDtypeStruct((2, T), jnp.int32),
            jax.ShapeDtypeStruct((1, NB), jnp.int32),
            jax.ShapeDtypeStruct((1, 1), jnp.int32),
            jax.ShapeDtypeStruct((1, 1), jnp.float32),
            jax.ShapeDtypeStruct((NR, D), jnp.bfloat16),
        ],
    )(x_flat, gate_w)

    sarr = jnp.concatenate([bexp.reshape(NB), nlive.reshape(1)])

    FC = min(1024, DFF)
    NC = DFF // FC
    yg = pl.pallas_call(
        _grouped_kernel,
        grid_spec=pltpu.PrefetchScalarGridSpec(
            num_scalar_prefetch=1,
            grid=(NC, NB),
            in_specs=[
                pl.BlockSpec((BT, D), lambda c, b, sarr: (b, 0)),
                pl.BlockSpec((1, FC, D), lambda c, b, sarr: (sarr[b], c, 0)),
                pl.BlockSpec((1, FC, D), lambda c, b, sarr: (sarr[b], c, 0)),
                pl.BlockSpec((1, D, FC), lambda c, b, sarr: (sarr[b], 0, c)),
            ],
            out_specs=pl.BlockSpec((NR, D), lambda c, b, sarr: (0, 0)),
            scratch_shapes=[
                pltpu.VMEM((FC, D), jnp.bfloat16),
                pltpu.VMEM((FC, D), jnp.bfloat16),
                pltpu.VMEM((D, FC), jnp.bfloat16),
            ],
        ),
        out_shape=jax.ShapeDtypeStruct((NR, D), jnp.float32),
        compiler_params=pltpu.CompilerParams(
            dimension_semantics=("arbitrary", "arbitrary")),
    )(sarr, xg, ew1, ew2, ew3)

    g = _dispatch_gather(yg, dest.reshape(2 * T))

    wcols = jnp.pad(ws.T, ((0, 0), (0, 126)))
    xb16 = x_flat.astype(jnp.bfloat16)

    TH = T // 2
    FCS = min(512, DFH)
    NCS = DFH // FCS
    out = pl.pallas_call(
        _shared_combine_kernel,
        grid=(2, NS, NCS),
        in_specs=[
            pl.BlockSpec((TH, D), lambda h, j, c: (h, 0)),
            pl.BlockSpec((1, FCS, D), lambda h, j, c: (j, c, 0)),
            pl.BlockSpec((1, FCS, D), lambda h, j, c: (j, c, 0)),
            pl.BlockSpec((1, D, FCS), lambda h, j, c: (j, 0, c)),
            pl.BlockSpec((TH, D), lambda h, j, c: (h, 0)),
            pl.BlockSpec((TH, D), lambda h, j, c: (h + 2, 0)),
            pl.BlockSpec((TH, 128), lambda h, j, c: (h, 0)),
        ],
        out_specs=pl.BlockSpec((TH, D), lambda h, j, c: (h, 0)),
        out_shape=jax.ShapeDtypeStruct((T, D), jnp.float32),
        compiler_params=pltpu.CompilerParams(
            dimension_semantics=("parallel", "arbitrary", "arbitrary")),
    )(xb16, sw1, sw2, sw3, g, g, wcols)

    return out.reshape(B, S, D), loss[0, 0]
